# Initial kernel scaffold; baseline (speedup 1.0000x reference)
import functools

import jax
import jax.numpy as jnp
from jax import lax
from jax.experimental import pallas as pl
from jax.experimental.pallas import tpu as pltpu
from jax.experimental.pallas import tpu_sc as plsc

_NC = 2
_NS = 16
_NW = _NC * _NS
_GROUP = 128


@functools.lru_cache(maxsize=None)
def _make_lookup(n, v, d):
    assert n % (_NW * _GROUP) == 0
    b_per_w = n // _NW
    groups_per_w = b_per_w // _GROUP

    max_rows = (320 * 1024) // (4 * d)
    g_chunk = 1
    for g in range(1, min(groups_per_w, 24) + 1):
        if groups_per_w % g == 0 and g * _GROUP <= max_rows:
            g_chunk = g
    C = g_chunk * _GROUP
    n_chunks = b_per_w // C

    mesh = plsc.VectorSubcoreMesh(core_axis_name="c", subcore_axis_name="s")

    @functools.partial(
        pl.kernel,
        mesh=mesh,
        out_type=jax.ShapeDtypeStruct((n, d), jnp.float32),
        scratch_types=[
            pltpu.VMEM((g_chunk, _GROUP), jnp.int32),
            pltpu.VMEM((C, d), jnp.float32),
            pltpu.SemaphoreType.DMA,
        ],
    )
    def emb_kernel(idx_hbm, tab_hbm, out_hbm, idx_v, rows_v, sem):
        wid = lax.axis_index("s") * _NC + lax.axis_index("c")
        base = wid * b_per_w
        grow0 = wid * groups_per_w

        def body(ci, carry):
            pltpu.sync_copy(idx_hbm.at[pl.ds(grow0 + ci * g_chunk, g_chunk)],
                            idx_v)
            copies = [
                pltpu.async_copy(tab_hbm.at[idx_v.at[g]],
                                 rows_v.at[pl.ds(g * _GROUP, _GROUP)],
                                 sem)
                for g in range(g_chunk)
            ]
            for cpy in copies:
                cpy.wait()
            pltpu.sync_copy(rows_v, out_hbm.at[pl.ds(base + ci * C, C)])
            return carry

        lax.fori_loop(0, n_chunks, body, 0)

    return emb_kernel


def kernel(token_id, table):
    b, l = token_id.shape
    v, d = table.shape
    n = b * l
    idx2d = token_id.reshape(n // _GROUP, _GROUP).astype(jnp.int32)
    out = _make_lookup(n, v, d)(idx2d, table)
    return out.reshape(b, l, d)


# speedup vs baseline: 1.4907x; 1.4907x over previous
import functools

import jax
import jax.numpy as jnp
from jax import lax
from jax.experimental import pallas as pl
from jax.experimental.pallas import tpu as pltpu
from jax.experimental.pallas import tpu_sc as plsc

_NC = 2
_NS = 16
_NW = _NC * _NS
_GROUP = 128


@functools.lru_cache(maxsize=None)
def _make_lookup(n, v, d):
    assert n % (_NW * _GROUP) == 0
    b_per_w = n // _NW
    groups_per_w = b_per_w // _GROUP

    max_rows = (320 * 1024) // (4 * d)
    g_chunk = 1
    for g in range(1, min(groups_per_w, 24) + 1):
        if groups_per_w % g == 0 and g * _GROUP <= max_rows:
            g_chunk = g
    C = g_chunk * _GROUP
    n_chunks = b_per_w // C

    mesh = plsc.VectorSubcoreMesh(core_axis_name="c", subcore_axis_name="s")

    @functools.partial(
        pl.kernel,
        mesh=mesh,
        out_type=jax.ShapeDtypeStruct((n, d), jnp.float32),
        scratch_types=[
            pltpu.VMEM((g_chunk, _GROUP), jnp.int32),
            pltpu.VMEM((C, d), jnp.float32),
            pltpu.SemaphoreType.DMA,
        ],
        compiler_params=pltpu.CompilerParams(use_tc_tiling_on_sc=False),
    )
    def emb_kernel(idx_hbm, tab_hbm, out_hbm, idx_v, rows_v, sem):
        wid = lax.axis_index("s") * _NC + lax.axis_index("c")
        base = wid * b_per_w

        def body(ci, carry):
            pltpu.sync_copy(idx_hbm.at[wid * n_chunks + ci], idx_v)
            copies = [
                pltpu.async_copy(tab_hbm.at[idx_v.at[g]],
                                 rows_v.at[pl.ds(g * _GROUP, _GROUP)],
                                 sem)
                for g in range(g_chunk)
            ]
            for cpy in copies:
                cpy.wait()
            pltpu.sync_copy(rows_v, out_hbm.at[pl.ds(base + ci * C, C)])
            return carry

        lax.fori_loop(0, n_chunks, body, 0)

    return emb_kernel, g_chunk


def kernel(token_id, table):
    b, l = token_id.shape
    v, d = table.shape
    n = b * l
    lookup, g_chunk = _make_lookup(n, v, d)
    idx3d = token_id.reshape(-1, g_chunk, _GROUP).astype(jnp.int32)
    out = lookup(idx3d, table)
    return out.reshape(b, l, d)
